# baseline (speedup 1.0000x reference)
import functools

import jax
import jax.numpy as jnp
from jax import lax
from jax.experimental import pallas as pl
from jax.experimental.pallas import tpu as pltpu
from jax.experimental.pallas import tpu_sc as plsc

B = 16384
L = 50
D = 64
N = B * L
NC = 2
NS = 16
NW = NC * NS
N_PER_W = N // NW
CHUNK = 128
STEPS = N_PER_W // CHUNK


def _emb_body(words_hbm, classes_hbm, noun_hbm, cls_hbm, out_hbm,
              widx, cidx, nbuf, cbuf, sem):
    wid = lax.axis_index("s") * NC + lax.axis_index("c")
    base = wid * N_PER_W

    def step(i, _):
        off = base + i * CHUNK
        pltpu.sync_copy(words_hbm.at[pl.ds(off, CHUNK)], widx)
        pltpu.sync_copy(classes_hbm.at[pl.ds(off, CHUNK)], cidx)
        pltpu.async_copy(noun_hbm.at[widx], nbuf, sem).wait()
        pltpu.async_copy(cls_hbm.at[cidx], cbuf, sem).wait()

        def add_row(r, _):
            for j in range(D // 16):
                sl = pl.ds(j * 16, 16)
                plsc.addupdate(nbuf.at[r, sl], cbuf[r, sl])
            return 0

        lax.fori_loop(0, CHUNK, add_row, 0)
        pltpu.sync_copy(nbuf, out_hbm.at[pl.ds(off, CHUNK)])
        return 0

    lax.fori_loop(0, STEPS, step, 0)


@jax.jit
def _emb(words_flat, classes_flat, noun_table, class_table):
    mesh = plsc.VectorSubcoreMesh(core_axis_name="c", subcore_axis_name="s")
    f = pl.kernel(
        _emb_body,
        out_type=jax.ShapeDtypeStruct((N, D), jnp.float32),
        mesh=mesh,
        scratch_types=[
            pltpu.VMEM((CHUNK,), jnp.int32),
            pltpu.VMEM((CHUNK,), jnp.int32),
            pltpu.VMEM((CHUNK, D), jnp.float32),
            pltpu.VMEM((CHUNK, D), jnp.float32),
            pltpu.SemaphoreType.DMA,
        ],
        compiler_params=pltpu.CompilerParams(use_tc_tiling_on_sc=False),
    )
    return f(words_flat, classes_flat, noun_table, class_table)


def kernel(words, classes, noun_table, class_table):
    out = _emb(words.reshape(N), classes.reshape(N), noun_table, class_table)
    return out.reshape(B, L, D)
